# baseline (speedup 1.0000x reference)
import functools

import jax
import jax.numpy as jnp
from jax import lax
from jax.experimental import pallas as pl
from jax.experimental.pallas import tpu as pltpu
from jax.experimental.pallas import tpu_sc as plsc

NUM_USERS = 1000000
NUM_ITEMS = 1000000
EMB = 32
B, L = 4096, 50
N = B * L
CH = 128
NC, NS = 2, 16
NW = NC * NS
CPT = N // (NW * CH)
PR = 16
KG = 10
GPR = 128 // EMB
CROWS = 1024
N_BLK = N * EMB // (CROWS * 128)
TBW = 8192


def _scatter_pos_body(item_hbm, pval_hbm, pos_hbm, idx_v, src_v, sem):
    wid = lax.axis_index("s") * NC + lax.axis_index("c")
    pltpu.sync_copy(item_hbm.at[wid], idx_v)

    def group(g, carry):
        pltpu.sync_copy(pval_hbm.at[wid, pl.ds(g * KG, KG)], src_v)
        for j in range(KG):
            pltpu.async_copy(
                src_v.at[j], pos_hbm.at[idx_v.at[g * KG + j]], sem
            )
        for j in range(KG):
            pltpu.make_async_copy(
                src_v.at[j], pos_hbm.at[idx_v.at[g * KG + j]], sem
            ).wait()
        return carry

    lax.fori_loop(0, CPT // KG, group, 0)


def _gather_body(user_hbm, item_hbm, item16_hbm, pos_hbm, users_hbm, ic_hbm,
                 it_hbm, ue_hbm, ice_hbm, ite_hbm, q_hbm,
                 uidx_v, iidx_v, i16_v, ubuf, cbuf, tbuf, qacc_v, gsem):
    wid = lax.axis_index("s") * NC + lax.axis_index("c")
    base = wid * CPT
    pltpu.sync_copy(user_hbm.at[wid], uidx_v)
    pltpu.sync_copy(item_hbm.at[wid], iidx_v)
    pltpu.sync_copy(item16_hbm.at[wid], i16_v)

    def chunk(j, carry):
        cu = pltpu.async_copy(users_hbm.at[uidx_v.at[j]], ubuf, gsem)
        cc = pltpu.async_copy(ic_hbm.at[iidx_v.at[j]], cbuf, gsem)
        ct = pltpu.async_copy(it_hbm.at[iidx_v.at[j]], tbuf, gsem)
        cq = pltpu.async_copy(
            pos_hbm.at[i16_v.at[j]], qacc_v.at[pl.ds(j * CH, CH)], gsem
        )
        cu.wait()
        cc.wait()
        ct.wait()
        cq.wait()
        row0 = (base + j) * CH
        pltpu.sync_copy(ubuf, ue_hbm.at[pl.ds(row0, CH)])
        pltpu.sync_copy(cbuf, ice_hbm.at[pl.ds(row0, CH)])
        pltpu.sync_copy(tbuf, ite_hbm.at[pl.ds(row0, CH)])
        return carry

    lax.fori_loop(0, CPT, chunk, 0)
    pltpu.sync_copy(qacc_v, q_hbm.at[wid])


@functools.cache
def _sc_kernels():
    mesh = plsc.VectorSubcoreMesh(
        core_axis_name="c", subcore_axis_name="s", num_cores=NC, num_subcores=NS
    )
    params = pltpu.CompilerParams(use_tc_tiling_on_sc=False)
    scatter_pos = pl.kernel(
        _scatter_pos_body,
        out_type=jax.ShapeDtypeStruct((NUM_ITEMS, PR), jnp.int32),
        mesh=mesh,
        scratch_types=[
            pltpu.VMEM((CPT, CH), jnp.int32),
            pltpu.VMEM((KG, CH, PR), jnp.int32),
            pltpu.SemaphoreType.DMA,
        ],
        name="sc_scatter_pos",
        compiler_params=params,
    )
    gather = pl.kernel(
        _gather_body,
        out_type=(
            jax.ShapeDtypeStruct((N, EMB), jnp.float32),
            jax.ShapeDtypeStruct((N, EMB), jnp.float32),
            jax.ShapeDtypeStruct((N, EMB), jnp.float32),
            jax.ShapeDtypeStruct((NW, CPT * CH), jnp.int32),
        ),
        mesh=mesh,
        scratch_types=[
            pltpu.VMEM((CPT, CH), jnp.int32),
            pltpu.VMEM((CPT, CH), jnp.int32),
            pltpu.VMEM((CPT, CH), jnp.int32),
            pltpu.VMEM((CH, EMB), jnp.float32),
            pltpu.VMEM((CH, EMB), jnp.float32),
            pltpu.VMEM((CH, EMB), jnp.float32),
            pltpu.VMEM((CPT * CH,), jnp.int32),
            pltpu.SemaphoreType.DMA,
        ],
        name="sc_gather",
        compiler_params=params,
    )
    return scatter_pos, gather


def _transpose_body(ut_ref, ct_ref, tt_ref, u_out, c_out, t_out):
    u_out[...] = ut_ref[...].T
    c_out[...] = ct_ref[...].T
    t_out[...] = tt_ref[...].T


_N_TBLK = -(-NUM_ITEMS // TBW)
_transpose = pl.pallas_call(
    _transpose_body,
    grid=(_N_TBLK,),
    in_specs=[pl.BlockSpec((EMB, TBW), lambda i: (0, i))] * 3,
    out_specs=[pl.BlockSpec((TBW, EMB), lambda i: (i, 0))] * 3,
    out_shape=[jax.ShapeDtypeStruct((NUM_ITEMS, EMB), jnp.float32)] * 3,
)


def _compute_body(ue_ref, ice_ref, ite_ref, lab_ref, w_ref, q_ref, *outs):
    i = pl.program_id(0)
    u = ue_ref[0]
    c = ice_ref[0]
    t = ite_ref[0]
    lab = lab_ref[...]
    w = w_ref[...]
    q = q_ref[...]

    gmat = (
        lax.broadcasted_iota(jnp.int32, (128, GPR), 0) // EMB
        == lax.broadcasted_iota(jnp.int32, (128, GPR), 1)
    ).astype(jnp.float32)

    def rowdot_t(x):
        return lax.dot_general(gmat, x, (((0,), (1,)), ((), ())))

    sc = rowdot_t(u * c)
    st = rowdot_t(u * t)
    d = c - t
    s = rowdot_t(d * d)

    j_iota = lax.broadcasted_iota(jnp.int32, (GPR, CROWS), 0)
    r_iota = lax.broadcasted_iota(jnp.int32, (GPR, CROWS), 1)
    pidx = (i * CROWS + r_iota) * GPR + j_iota
    winf = (q == pidx).astype(jnp.float32)
    nw = 1.0 - w

    def bce(x):
        return jnp.maximum(x, 0.0) - x * lab + jnp.log1p(jnp.exp(-jnp.abs(x)))

    sig = lambda x: 1.0 / (1.0 + jnp.exp(-x))
    sums = (
        jnp.sum(bce(sc) * nw),
        jnp.sum(bce(st) * w),
        jnp.sum(jnp.abs(sig(sc) - lab) * nw),
        jnp.sum(jnp.abs(sig(st) - lab) * w),
        jnp.sum(w),
        jnp.sum(s * winf),
        jnp.sum(winf),
    )
    for o_ref, val in zip(outs, sums):
        @pl.when(i == 0)
        def _init(o_ref=o_ref):
            o_ref[...] = jnp.zeros_like(o_ref)

        o_ref[...] += val


_N_SUMS = 7
_compute = pl.pallas_call(
    _compute_body,
    grid=(N_BLK,),
    in_specs=[
        pl.BlockSpec((1, CROWS, 128), lambda i: (i, 0, 0)),
        pl.BlockSpec((1, CROWS, 128), lambda i: (i, 0, 0)),
        pl.BlockSpec((1, CROWS, 128), lambda i: (i, 0, 0)),
        pl.BlockSpec((GPR, CROWS), lambda i: (0, i)),
        pl.BlockSpec((GPR, CROWS), lambda i: (0, i)),
        pl.BlockSpec((GPR, CROWS), lambda i: (0, i)),
    ],
    out_specs=[pl.BlockSpec((1, 128), lambda i: (0, 0))] * _N_SUMS,
    out_shape=[jax.ShapeDtypeStruct((1, 128), jnp.float32)] * _N_SUMS,
)


def kernel(user, item, label, mask, users, items_control, items_treatment):
    user3d = user.reshape(NW, CPT, CH)
    item3d = item.reshape(NW, CPT, CH)
    pval4d = jnp.broadcast_to(
        jnp.arange(N, dtype=jnp.int32)[:, None], (N, PR)
    ).reshape(NW, CPT, CH, PR)
    item16_3d = (item * PR).reshape(NW, CPT, CH)

    users_rm, ic_rm, it_rm = _transpose(
        users.T, items_control.T, items_treatment.T
    )

    _scatter_pos, _gather = _sc_kernels()
    pos = _scatter_pos(item3d, pval4d)
    ue, ice, ite, q2d = _gather(
        user3d, item3d, item16_3d, pos.reshape(NUM_ITEMS * PR),
        users_rm, ic_rm, it_rm
    )

    lab_t = label.reshape(N // GPR, GPR).T
    w_t = jnp.broadcast_to(
        mask.astype(jnp.float32)[:, None], (B, L)
    ).reshape(N // GPR, GPR).T
    q_t = q2d.reshape(N // GPR, GPR).T

    sums = _compute(
        ue.reshape(N_BLK, CROWS, 128),
        ice.reshape(N_BLK, CROWS, 128),
        ite.reshape(N_BLK, CROWS, 128),
        lab_t, w_t, q_t,
    )
    s_bce_c, s_bce_t, s_dc, s_dt, s_w, s_sw, s_win = (o[0, 0] for o in sums)

    seq_len = jnp.float32(L)
    cnt_t = s_w / seq_len
    cnt_c = jnp.float32(B) - cnt_t
    control_loss = s_bce_c / (cnt_c * seq_len)
    treatment_loss = s_bce_t / (cnt_t * seq_len)
    control_distance = s_dc / (cnt_c * seq_len)
    treatment_distance = s_dt / (cnt_t * seq_len)
    discrepancy_loss = s_sw / (s_win * jnp.float32(EMB))
    return (control_loss, treatment_loss, discrepancy_loss,
            control_distance, treatment_distance)
